# baseline (speedup 1.0000x reference)
import functools

import jax
import jax.numpy as jnp
from jax import lax
from jax.experimental import pallas as pl
from jax.experimental.pallas import tpu as pltpu
from jax.experimental.pallas import tpu_sc as plsc

N = 10000
E = 320000
F_IN = 128
HID = 30
EPS = 1e-5

NC = 2
NS = 16
NW = NC * NS
CHUNK = 128
EDGES_PER_W = 10240
E_PAD = NW * EDGES_PER_W
CH_PER_W = EDGES_PER_W // CHUNK
GRP = 4
N_PAD = 10240
ROWS_PER_TILE = N_PAD // NS

_mesh = plsc.VectorSubcoreMesh(core_axis_name="c", subcore_axis_name="s")


CH0 = 80
CH1 = 160 - CH0
CH_MAX = max(CH0, CH1)


def _make_prop(width):
    @functools.partial(
        pl.kernel,
        mesh=_mesh,
        compiler_params=pltpu.CompilerParams(use_tc_tiling_on_sc=False),
        out_type=jax.ShapeDtypeStruct((NC, N_PAD, width), jnp.float32),
        scratch_types=[
            pltpu.VMEM((CH_MAX + GRP, CHUNK), jnp.int32),
            pltpu.VMEM((CH_MAX + GRP, CHUNK), jnp.int32),
            pltpu.VMEM((2 * GRP, CHUNK, width), jnp.float32),
            pltpu.VMEM((16, width), jnp.float32),
            pltpu.VMEM_SHARED((N_PAD, width), jnp.float32),
            pltpu.SemaphoreType.DMA,
            pltpu.SemaphoreType.DMA,
            pltpu.SemaphoreType.DMA,
            pltpu.SemaphoreType.DMA,
        ],
    )
    def k(u_hbm, src_hbm, dst_hbm, out_hbm, sidx, didx, gbuf, zbuf, acc,
          gsem0, gsem1, ssem0, ssem1):
        cid = lax.axis_index("c")
        sid = lax.axis_index("s")
        base = sid * ROWS_PER_TILE
        ch = jnp.where(cid == 0, CH0, CH1)
        row0 = jnp.where(cid == 0, sid * CH0, NS * CH0 + sid * CH1)
        ngrp = ch // GRP
        gsems = (gsem0, gsem1)
        ssems = (ssem0, ssem1)

        for i in range(16):
            for c in range(width // 16):
                zbuf[i, pl.ds(c * 16, 16)] = jnp.zeros((16,), jnp.float32)

        nidx = CH_MAX + GRP
        tail = nidx // 2
        prow = row0 // 2

        def zero_fire(i, carry):
            pltpu.async_copy(zbuf, acc.at[pl.ds(base + i * 16, 16)], gsem0)
            return carry

        lax.fori_loop(0, ROWS_PER_TILE // 16, zero_fire, 0)
        pltpu.async_copy(src_hbm.at[pl.ds(prow, tail)],
                         sidx.at[pl.ds(tail, tail)], gsem1)
        pltpu.async_copy(dst_hbm.at[pl.ds(prow, tail)],
                         didx.at[pl.ds(tail, tail)], ssem0)

        def zero_drain(i, carry):
            pltpu.make_async_copy(zbuf, acc.at[pl.ds(base + i * 16, 16)],
                                  gsem0).wait()
            return carry

        lax.fori_loop(0, ROWS_PER_TILE // 16, zero_drain, 0)
        pltpu.make_async_copy(src_hbm.at[pl.ds(prow, tail)],
                              sidx.at[pl.ds(tail, tail)], gsem1).wait()
        pltpu.make_async_copy(dst_hbm.at[pl.ds(prow, tail)],
                              didx.at[pl.ds(tail, tail)], ssem0).wait()

        def unpack_row(ref, j):
            half = (j % 2) * (CHUNK // 2)
            srcrow = tail + j // 2
            words = []
            for c in range(CHUNK // 32):
                words.append(ref[srcrow, pl.ds(half + c * 16, 16)])
            for c, w in enumerate(words):
                lo = jnp.bitwise_and(w, 0xFFFF)
                hi = lax.shift_right_logical(w, 16)
                ref[j, pl.ds(c * 32, 16)] = lo
                ref[j, pl.ds(c * 32 + 16, 16)] = hi

        def unpack_body(j, carry):
            unpack_row(sidx, j)
            unpack_row(didx, j)
            return carry

        lax.fori_loop(0, nidx, unpack_body, 0)
        plsc.subcore_barrier()

        def gather(g, pset, sem):
            for b in range(GRP):
                pltpu.async_copy(u_hbm.at[sidx.at[g * GRP + b]],
                                 gbuf.at[pset * GRP + b], sem)

        def gather_wait(g, pset, sem):
            for b in range(GRP):
                pltpu.make_async_copy(u_hbm.at[sidx.at[g * GRP + b]],
                                      gbuf.at[pset * GRP + b], sem).wait()

        def scatter(g, pset, sem):
            for b in range(GRP):
                pltpu.async_copy(gbuf.at[pset * GRP + b],
                                 acc.at[didx.at[g * GRP + b]], sem, add=True)

        def scatter_wait(g, pset, sem):
            for b in range(GRP):
                pltpu.make_async_copy(gbuf.at[pset * GRP + b],
                                      acc.at[didx.at[g * GRP + b]],
                                      sem).wait()

        gather(0, 0, gsems[0])
        gather_wait(0, 0, gsems[0])
        scatter(0, 0, ssems[0])
        gather(1, 1, gsems[1])
        gather_wait(1, 1, gsems[1])
        scatter(1, 1, ssems[1])
        scatter_wait(0, 0, ssems[0])
        gather(2, 0, gsems[0])

        def pipe_body(t, carry):
            g0 = 2 * t
            gather_wait(g0, 0, gsems[0])
            scatter(g0, 0, ssems[0])
            scatter_wait(g0 - 1, 1, ssems[1])
            gather(g0 + 1, 1, gsems[1])
            g1 = g0 + 1
            gather_wait(g1, 1, gsems[1])
            scatter(g1, 1, ssems[1])
            scatter_wait(g1 - 1, 0, ssems[0])
            gather(g1 + 1, 0, gsems[0])
            return carry

        lax.fori_loop(1, ngrp // 2, pipe_body, 0)
        gather_wait(ngrp, 0, gsems[0])
        scatter_wait(ngrp - 1, 1, ssems[1])
        plsc.subcore_barrier()
        pltpu.sync_copy(acc.at[pl.ds(base, ROWS_PER_TILE)],
                        out_hbm.at[cid, pl.ds(base, ROWS_PER_TILE)])

    return k


_prop64 = _make_prop(64)
_prop32 = _make_prop(32)


@functools.partial(
    pl.kernel,
    mesh=_mesh,
    compiler_params=pltpu.CompilerParams(use_tc_tiling_on_sc=False),
    out_type=jax.ShapeDtypeStruct((NC, N_PAD), jnp.float32),
    scratch_types=[
        pltpu.VMEM((CH_PER_W, CHUNK), jnp.int32),
        pltpu.VMEM((CHUNK,), jnp.float32),
        pltpu.VMEM((ROWS_PER_TILE,), jnp.float32),
        pltpu.VMEM_SHARED((N_PAD,), jnp.float32),
        pltpu.SemaphoreType.DMA,
    ],
)
def _deg_kernel(src_hbm, out_hbm, sidx, ones, zbuf, acc, ssem):
    cid = lax.axis_index("c")
    sid = lax.axis_index("s")
    w = cid * NS + sid
    base = sid * ROWS_PER_TILE

    for i in range(CHUNK // 16):
        ones[pl.ds(i * 16, 16)] = jnp.ones((16,), jnp.float32)
    for i in range(ROWS_PER_TILE // 16):
        zbuf[pl.ds(i * 16, 16)] = jnp.zeros((16,), jnp.float32)
    pltpu.sync_copy(zbuf, acc.at[pl.ds(base, ROWS_PER_TILE)])
    pltpu.sync_copy(src_hbm.at[pl.ds(w * CH_PER_W, CH_PER_W)], sidx)
    plsc.subcore_barrier()

    def grp_body(g, carry):
        ss = []
        for b in range(GRP):
            j = g * GRP + b
            ss.append(pltpu.async_copy(ones, acc.at[sidx.at[j]], ssem,
                                       add=True))
        for s in ss:
            s.wait()
        return carry

    lax.fori_loop(0, CH_PER_W // GRP, grp_body, 0)
    plsc.subcore_barrier()
    pltpu.sync_copy(acc.at[pl.ds(base, ROWS_PER_TILE)],
                    out_hbm.at[cid, pl.ds(base, ROWS_PER_TILE)])


_TC_PARAMS = pltpu.CompilerParams(vmem_limit_bytes=100 * 1024 * 1024)


def _tc0_body(x_ref, wc_ref, o_y):
    o_y[...] = jnp.dot(x_ref[...], wc_ref[...],
                       preferred_element_type=jnp.float32,
                       precision=lax.Precision.HIGHEST)


_tc0 = pl.pallas_call(
    _tc0_body,
    compiler_params=_TC_PARAMS,
    out_shape=jax.ShapeDtypeStruct((N, 3 * HID), jnp.float32),
)


def _tc1_body(degp_ref, y_ref, o_dinv, o_y0, o_u):
    deg = degp_ref[0, :N] + degp_ref[1, :N]
    dinv = jnp.where(deg > 0, 1.0 / jnp.sqrt(deg), 0.0)
    o_dinv[...] = dinv[:, None]
    y = y_ref[...]
    o_y0[...] = y[:, 0:HID]
    d = dinv[:, None]
    z2 = jnp.zeros((N, 2), jnp.float32)
    o_u[...] = jnp.concatenate(
        [d * y[:, HID:2 * HID], z2, d * y[:, 2 * HID:3 * HID], z2], axis=1)


_tc1 = pl.pallas_call(
    _tc1_body,
    compiler_params=_TC_PARAMS,
    out_shape=(
        jax.ShapeDtypeStruct((N, 1), jnp.float32),
        jax.ShapeDtypeStruct((N, HID), jnp.float32),
        jax.ShapeDtypeStruct((N, 64), jnp.float32),
    ),
)


def _tc2_body(r_ref, dinv_ref, o_p1, o_v):
    r = r_ref[0] + r_ref[1]
    d = dinv_ref[...]
    o_p1[...] = -d * r[:N, 0:HID]
    v2 = -(d * d) * r[:N, 32:32 + HID]
    o_v[...] = jnp.concatenate([v2, jnp.zeros((N, 2), jnp.float32)], axis=1)


_tc2 = pl.pallas_call(
    _tc2_body,
    compiler_params=_TC_PARAMS,
    out_shape=(
        jax.ShapeDtypeStruct((N, HID), jnp.float32),
        jax.ShapeDtypeStruct((N, 32), jnp.float32),
    ),
)


def _combine_bn(r3_ref, y0_ref, p1_ref, dinv_ref, b_ref, g_ref, be_ref):
    r3 = r3_ref[0] + r3_ref[1]
    q2 = -dinv_ref[...] * r3[:N, 0:HID]
    pre = y0_ref[...] + p1_ref[...] + 2.0 * q2 + b_ref[...]
    h = jnp.maximum(pre, 0.0)
    mu = jnp.mean(h, axis=0, keepdims=True)
    var = jnp.mean((h - mu) ** 2, axis=0, keepdims=True)
    return g_ref[...] * (h - mu) / jnp.sqrt(var + EPS) + be_ref[...]


def _tc3_body(r3_ref, y0_ref, p1_ref, dinv_ref, b_ref, g_ref, be_ref,
              wc2_ref, o_y0b, o_u2):
    hb = _combine_bn(r3_ref, y0_ref, p1_ref, dinv_ref, b_ref, g_ref, be_ref)
    z = jnp.dot(hb, wc2_ref[...], preferred_element_type=jnp.float32,
                precision=lax.Precision.HIGHEST)
    o_y0b[...] = z[:, 0:HID]
    d = dinv_ref[...]
    z2 = jnp.zeros((N, 2), jnp.float32)
    o_u2[...] = jnp.concatenate(
        [d * z[:, HID:2 * HID], z2, d * z[:, 2 * HID:3 * HID], z2], axis=1)


_tc3 = pl.pallas_call(
    _tc3_body,
    compiler_params=_TC_PARAMS,
    out_shape=(
        jax.ShapeDtypeStruct((N, HID), jnp.float32),
        jax.ShapeDtypeStruct((N, 64), jnp.float32),
    ),
)


def _tc5_body(r3_ref, y0_ref, p1_ref, dinv_ref, b_ref, g_ref, be_ref, o_out):
    o_out[...] = _combine_bn(r3_ref, y0_ref, p1_ref, dinv_ref, b_ref, g_ref,
                             be_ref)


_tc5 = pl.pallas_call(
    _tc5_body,
    compiler_params=_TC_PARAMS,
    out_shape=jax.ShapeDtypeStruct((N, HID), jnp.float32),
)


def _pad_rows(a):
    return jnp.pad(a, ((0, N_PAD - N), (0, 0)))


def kernel(x, edge_index, batch, W1, b1, W2, b2, gamma1, beta1, gamma2,
           beta2):
    src = edge_index[0]
    dst = edge_index[1]
    n_rows = E_PAD // CHUNK + GRP
    fill = jnp.full((n_rows * CHUNK - E,), N, jnp.int32)
    srcf = jnp.concatenate([src, fill])
    dstf = jnp.concatenate([dst, fill])
    srcp = srcf.reshape(n_rows, CHUNK)

    def _pack(flat):
        pairs = flat.reshape(-1, 2)
        return (pairs[:, 0] | (pairs[:, 1] << 16)).reshape(
            n_rows // 2, CHUNK)

    srcpk = _pack(srcf)
    dstpk = _pack(dstf)

    degp = _deg_kernel(srcp)

    wc1 = jnp.concatenate([W1[0] - W1[2], W1[1], W1[2]], axis=1)
    wc2 = jnp.concatenate([W2[0] - W2[2], W2[1], W2[2]], axis=1)
    b1r = b1[None, :]
    b2r = b2[None, :]
    g1r = gamma1[None, :]
    g2r = gamma2[None, :]
    be1r = beta1[None, :]
    be2r = beta2[None, :]

    y_proj = _tc0(x, wc1)
    dinv, y0, u = _tc1(degp, y_proj)
    r = _prop64(_pad_rows(u), srcpk, dstpk)
    p1, v = _tc2(r, dinv)
    r3 = _prop32(_pad_rows(v), srcpk, dstpk)
    y0b, u2 = _tc3(r3, y0, p1, dinv, b1r, g1r, be1r, wc2)
    r_2 = _prop64(_pad_rows(u2), srcpk, dstpk)
    p1b, v_2 = _tc2(r_2, dinv)
    r3b = _prop32(_pad_rows(v_2), srcpk, dstpk)
    return _tc5(r3b, y0b, p1b, dinv, b2r, g2r, be2r)


# speedup vs baseline: 1.7099x; 1.7099x over previous
import functools

import jax
import jax.numpy as jnp
from jax import lax
from jax.experimental import pallas as pl
from jax.experimental.pallas import tpu as pltpu
from jax.experimental.pallas import tpu_sc as plsc

N = 10000
E = 320000
F_IN = 128
HID = 30
EPS = 1e-5

NC = 2
NS = 16
NW = NC * NS
CHUNK = 128
EDGES_PER_W = 10240
E_PAD = NW * EDGES_PER_W
CH_PER_W = EDGES_PER_W // CHUNK
GRP = 4
N_PAD = 10240
ROWS_PER_TILE = N_PAD // NS

_mesh = plsc.VectorSubcoreMesh(core_axis_name="c", subcore_axis_name="s")


CH0 = 80
CH1 = 160 - CH0
CH_MAX = max(CH0, CH1)


def _make_prop(width):
    @functools.partial(
        pl.kernel,
        mesh=_mesh,
        compiler_params=pltpu.CompilerParams(use_tc_tiling_on_sc=False),
        out_type=jax.ShapeDtypeStruct((NC, N_PAD, width), jnp.float32),
        scratch_types=[
            pltpu.VMEM((CH_MAX + GRP, CHUNK), jnp.int32),
            pltpu.VMEM((CH_MAX + GRP, CHUNK), jnp.int32),
            pltpu.VMEM((2 * GRP, CHUNK, width), jnp.float32),
            pltpu.VMEM((16, width), jnp.float32),
            pltpu.VMEM_SHARED((N_PAD, width), jnp.float32),
            pltpu.SemaphoreType.DMA,
            pltpu.SemaphoreType.DMA,
            pltpu.SemaphoreType.DMA,
            pltpu.SemaphoreType.DMA,
        ],
    )
    def k(u_hbm, src_hbm, dst_hbm, out_hbm, sidx, didx, gbuf, zbuf, acc,
          gsem0, gsem1, ssem0, ssem1):
        cid = lax.axis_index("c")
        sid = lax.axis_index("s")
        base = sid * ROWS_PER_TILE
        ch = jnp.where(cid == 0, CH0, CH1)
        row0 = jnp.where(cid == 0, sid * CH0, NS * CH0 + sid * CH1)
        ngrp = ch // GRP
        gsems = (gsem0, gsem1)
        ssems = (ssem0, ssem1)

        for i in range(16):
            for c in range(width // 16):
                zbuf[i, pl.ds(c * 16, 16)] = jnp.zeros((16,), jnp.float32)

        def zero_fire(i, carry):
            pltpu.async_copy(zbuf, acc.at[pl.ds(base + i * 16, 16)], gsem0)
            return carry

        lax.fori_loop(0, ROWS_PER_TILE // 16, zero_fire, 0)
        pltpu.async_copy(src_hbm.at[pl.ds(row0, CH_MAX + GRP)], sidx, gsem1)
        pltpu.async_copy(dst_hbm.at[pl.ds(row0, CH_MAX + GRP)], didx, ssem0)

        def zero_drain(i, carry):
            pltpu.make_async_copy(zbuf, acc.at[pl.ds(base + i * 16, 16)],
                                  gsem0).wait()
            return carry

        lax.fori_loop(0, ROWS_PER_TILE // 16, zero_drain, 0)
        pltpu.make_async_copy(src_hbm.at[pl.ds(row0, CH_MAX + GRP)], sidx,
                              gsem1).wait()
        pltpu.make_async_copy(dst_hbm.at[pl.ds(row0, CH_MAX + GRP)], didx,
                              ssem0).wait()
        plsc.subcore_barrier()

        def gather(g, pset, sem):
            for b in range(GRP):
                pltpu.async_copy(u_hbm.at[sidx.at[g * GRP + b]],
                                 gbuf.at[pset * GRP + b], sem)

        def gather_wait(g, pset, sem):
            for b in range(GRP):
                pltpu.make_async_copy(u_hbm.at[sidx.at[g * GRP + b]],
                                      gbuf.at[pset * GRP + b], sem).wait()

        def scatter(g, pset, sem):
            for b in range(GRP):
                pltpu.async_copy(gbuf.at[pset * GRP + b],
                                 acc.at[didx.at[g * GRP + b]], sem, add=True)

        def scatter_wait(g, pset, sem):
            for b in range(GRP):
                pltpu.make_async_copy(gbuf.at[pset * GRP + b],
                                      acc.at[didx.at[g * GRP + b]],
                                      sem).wait()

        gather(0, 0, gsems[0])
        gather_wait(0, 0, gsems[0])
        scatter(0, 0, ssems[0])
        gather(1, 1, gsems[1])
        gather_wait(1, 1, gsems[1])
        scatter(1, 1, ssems[1])
        scatter_wait(0, 0, ssems[0])
        gather(2, 0, gsems[0])

        def pipe_body(t, carry):
            g0 = 2 * t
            gather_wait(g0, 0, gsems[0])
            scatter(g0, 0, ssems[0])
            scatter_wait(g0 - 1, 1, ssems[1])
            gather(g0 + 1, 1, gsems[1])
            g1 = g0 + 1
            gather_wait(g1, 1, gsems[1])
            scatter(g1, 1, ssems[1])
            scatter_wait(g1 - 1, 0, ssems[0])
            gather(g1 + 1, 0, gsems[0])
            return carry

        lax.fori_loop(1, ngrp // 2, pipe_body, 0)
        gather_wait(ngrp, 0, gsems[0])
        scatter_wait(ngrp - 1, 1, ssems[1])
        plsc.subcore_barrier()
        pltpu.sync_copy(acc.at[pl.ds(base, ROWS_PER_TILE)],
                        out_hbm.at[cid, pl.ds(base, ROWS_PER_TILE)])

    return k


_prop64 = _make_prop(64)
_prop32 = _make_prop(32)


@functools.partial(
    pl.kernel,
    mesh=_mesh,
    compiler_params=pltpu.CompilerParams(use_tc_tiling_on_sc=False),
    out_type=jax.ShapeDtypeStruct((NC, N_PAD), jnp.float32),
    scratch_types=[
        pltpu.VMEM((CH_PER_W, CHUNK), jnp.int32),
        pltpu.VMEM((CHUNK,), jnp.float32),
        pltpu.VMEM((ROWS_PER_TILE,), jnp.float32),
        pltpu.VMEM_SHARED((N_PAD,), jnp.float32),
        pltpu.SemaphoreType.DMA,
    ],
)
def _deg_kernel(src_hbm, out_hbm, sidx, ones, zbuf, acc, ssem):
    cid = lax.axis_index("c")
    sid = lax.axis_index("s")
    w = cid * NS + sid
    base = sid * ROWS_PER_TILE

    for i in range(CHUNK // 16):
        ones[pl.ds(i * 16, 16)] = jnp.ones((16,), jnp.float32)
    for i in range(ROWS_PER_TILE // 16):
        zbuf[pl.ds(i * 16, 16)] = jnp.zeros((16,), jnp.float32)
    pltpu.sync_copy(zbuf, acc.at[pl.ds(base, ROWS_PER_TILE)])
    pltpu.sync_copy(src_hbm.at[pl.ds(w * CH_PER_W, CH_PER_W)], sidx)
    plsc.subcore_barrier()

    def grp_body(g, carry):
        ss = []
        for b in range(GRP):
            j = g * GRP + b
            ss.append(pltpu.async_copy(ones, acc.at[sidx.at[j]], ssem,
                                       add=True))
        for s in ss:
            s.wait()
        return carry

    lax.fori_loop(0, CH_PER_W // GRP, grp_body, 0)
    plsc.subcore_barrier()
    pltpu.sync_copy(acc.at[pl.ds(base, ROWS_PER_TILE)],
                    out_hbm.at[cid, pl.ds(base, ROWS_PER_TILE)])


_TC_PARAMS = pltpu.CompilerParams(vmem_limit_bytes=100 * 1024 * 1024)


def _tc0_body(x_ref, wc_ref, o_y):
    o_y[...] = jnp.dot(x_ref[...], wc_ref[...],
                       preferred_element_type=jnp.float32,
                       precision=lax.Precision.HIGHEST)


_tc0 = pl.pallas_call(
    _tc0_body,
    compiler_params=_TC_PARAMS,
    out_shape=jax.ShapeDtypeStruct((N, 3 * HID), jnp.float32),
)


def _tc1_body(degp_ref, y_ref, o_dinv, o_y0, o_u):
    deg = degp_ref[0, :N] + degp_ref[1, :N]
    dinv = jnp.where(deg > 0, 1.0 / jnp.sqrt(deg), 0.0)
    o_dinv[...] = dinv[:, None]
    y = y_ref[...]
    o_y0[...] = y[:, 0:HID]
    d = dinv[:, None]
    z2 = jnp.zeros((N, 2), jnp.float32)
    o_u[...] = jnp.concatenate(
        [d * y[:, HID:2 * HID], z2, d * y[:, 2 * HID:3 * HID], z2], axis=1)


_tc1 = pl.pallas_call(
    _tc1_body,
    compiler_params=_TC_PARAMS,
    out_shape=(
        jax.ShapeDtypeStruct((N, 1), jnp.float32),
        jax.ShapeDtypeStruct((N, HID), jnp.float32),
        jax.ShapeDtypeStruct((N, 64), jnp.float32),
    ),
)


def _tc2_body(r_ref, dinv_ref, o_p1, o_v):
    r = r_ref[0] + r_ref[1]
    d = dinv_ref[...]
    o_p1[...] = -d * r[:N, 0:HID]
    v2 = -(d * d) * r[:N, 32:32 + HID]
    o_v[...] = jnp.concatenate([v2, jnp.zeros((N, 2), jnp.float32)], axis=1)


_tc2 = pl.pallas_call(
    _tc2_body,
    compiler_params=_TC_PARAMS,
    out_shape=(
        jax.ShapeDtypeStruct((N, HID), jnp.float32),
        jax.ShapeDtypeStruct((N, 32), jnp.float32),
    ),
)


def _combine_bn(r3_ref, y0_ref, p1_ref, dinv_ref, b_ref, g_ref, be_ref):
    r3 = r3_ref[0] + r3_ref[1]
    q2 = -dinv_ref[...] * r3[:N, 0:HID]
    pre = y0_ref[...] + p1_ref[...] + 2.0 * q2 + b_ref[...]
    h = jnp.maximum(pre, 0.0)
    mu = jnp.mean(h, axis=0, keepdims=True)
    var = jnp.mean((h - mu) ** 2, axis=0, keepdims=True)
    return g_ref[...] * (h - mu) / jnp.sqrt(var + EPS) + be_ref[...]


def _tc3_body(r3_ref, y0_ref, p1_ref, dinv_ref, b_ref, g_ref, be_ref,
              wc2_ref, o_y0b, o_u2):
    hb = _combine_bn(r3_ref, y0_ref, p1_ref, dinv_ref, b_ref, g_ref, be_ref)
    z = jnp.dot(hb, wc2_ref[...], preferred_element_type=jnp.float32,
                precision=lax.Precision.HIGHEST)
    o_y0b[...] = z[:, 0:HID]
    d = dinv_ref[...]
    z2 = jnp.zeros((N, 2), jnp.float32)
    o_u2[...] = jnp.concatenate(
        [d * z[:, HID:2 * HID], z2, d * z[:, 2 * HID:3 * HID], z2], axis=1)


_tc3 = pl.pallas_call(
    _tc3_body,
    compiler_params=_TC_PARAMS,
    out_shape=(
        jax.ShapeDtypeStruct((N, HID), jnp.float32),
        jax.ShapeDtypeStruct((N, 64), jnp.float32),
    ),
)


def _tc5_body(r3_ref, y0_ref, p1_ref, dinv_ref, b_ref, g_ref, be_ref, o_out):
    o_out[...] = _combine_bn(r3_ref, y0_ref, p1_ref, dinv_ref, b_ref, g_ref,
                             be_ref)


_tc5 = pl.pallas_call(
    _tc5_body,
    compiler_params=_TC_PARAMS,
    out_shape=jax.ShapeDtypeStruct((N, HID), jnp.float32),
)


def _pad_rows(a):
    return jnp.pad(a, ((0, N_PAD - N), (0, 0)))


def kernel(x, edge_index, batch, W1, b1, W2, b2, gamma1, beta1, gamma2,
           beta2):
    src = edge_index[0]
    dst = edge_index[1]
    n_rows = E_PAD // CHUNK + GRP
    fill = jnp.full((n_rows * CHUNK - E,), N, jnp.int32)
    srcp = jnp.concatenate([src, fill]).reshape(n_rows, CHUNK)
    dstp = jnp.concatenate([dst, fill]).reshape(n_rows, CHUNK)

    degp = _deg_kernel(srcp)

    wc1 = jnp.concatenate([W1[0] - W1[2], W1[1], W1[2]], axis=1)
    wc2 = jnp.concatenate([W2[0] - W2[2], W2[1], W2[2]], axis=1)
    b1r = b1[None, :]
    b2r = b2[None, :]
    g1r = gamma1[None, :]
    g2r = gamma2[None, :]
    be1r = beta1[None, :]
    be2r = beta2[None, :]

    y_proj = _tc0(x, wc1)
    dinv, y0, u = _tc1(degp, y_proj)
    r = _prop64(_pad_rows(u), srcp, dstp)
    p1, v = _tc2(r, dinv)
    r3 = _prop32(_pad_rows(v), srcp, dstp)
    y0b, u2 = _tc3(r3, y0, p1, dinv, b1r, g1r, be1r, wc2)
    r_2 = _prop64(_pad_rows(u2), srcp, dstp)
    p1b, v_2 = _tc2(r_2, dinv)
    r3b = _prop32(_pad_rows(v_2), srcp, dstp)
    return _tc5(r3b, y0b, p1b, dinv, b2r, g2r, be2r)


# speedup vs baseline: 1.8597x; 1.0876x over previous
import functools

import jax
import jax.numpy as jnp
from jax import lax
from jax.experimental import pallas as pl
from jax.experimental.pallas import tpu as pltpu
from jax.experimental.pallas import tpu_sc as plsc

N = 10000
E = 320000
F_IN = 128
HID = 30
EPS = 1e-5

NC = 2
NS = 16
NW = NC * NS
CHUNK = 128
E_PAD = 327680
GRP = 4
N_PAD = 10240
ROWS_PER_TILE = N_PAD // NS
CH_CS = E_PAD // NS // CHUNK
NGRP = CH_CS // GRP
CH_DEG = E_PAD // NW // CHUNK

_mesh = plsc.VectorSubcoreMesh(core_axis_name="c", subcore_axis_name="s")
_SC_PARAMS = pltpu.CompilerParams(use_tc_tiling_on_sc=False)


def _make_prop(width):
    @functools.partial(
        pl.kernel,
        mesh=_mesh,
        compiler_params=_SC_PARAMS,
        out_type=jax.ShapeDtypeStruct((NC, N_PAD, width), jnp.float32),
        scratch_types=[
            pltpu.VMEM((CH_CS + GRP, CHUNK), jnp.int32),
            pltpu.VMEM((CH_CS + GRP, CHUNK), jnp.int32),
            pltpu.VMEM((2 * GRP, CHUNK, width), jnp.float32),
            pltpu.VMEM((16, width), jnp.float32),
            pltpu.VMEM_SHARED((N_PAD, width), jnp.float32),
            pltpu.SemaphoreType.DMA,
            pltpu.SemaphoreType.DMA,
            pltpu.SemaphoreType.DMA,
            pltpu.SemaphoreType.DMA,
        ],
    )
    def k(u_hbm, src_hbm, dst_hbm, out_hbm, sidx, didx, gbuf, zbuf, acc,
          gsem0, gsem1, ssem0, ssem1):
        cid = lax.axis_index("c")
        sid = lax.axis_index("s")
        base = sid * ROWS_PER_TILE
        row0 = sid * CH_CS
        gsems = (gsem0, gsem1)
        ssems = (ssem0, ssem1)

        for i in range(16):
            for c in range(width // 16):
                zbuf[i, pl.ds(c * 16, 16)] = jnp.zeros((16,), jnp.float32)

        def zero_fire(i, carry):
            pltpu.async_copy(zbuf, acc.at[pl.ds(base + i * 16, 16)], gsem0)
            return carry

        lax.fori_loop(0, ROWS_PER_TILE // 16, zero_fire, 0)
        pltpu.async_copy(src_hbm.at[pl.ds(row0, CH_CS + GRP)], sidx, gsem1)
        pltpu.async_copy(dst_hbm.at[pl.ds(row0, CH_CS + GRP)], didx, ssem0)

        def zero_drain(i, carry):
            pltpu.make_async_copy(zbuf, acc.at[pl.ds(base + i * 16, 16)],
                                  gsem0).wait()
            return carry

        lax.fori_loop(0, ROWS_PER_TILE // 16, zero_drain, 0)
        pltpu.make_async_copy(src_hbm.at[pl.ds(row0, CH_CS + GRP)], sidx,
                              gsem1).wait()
        pltpu.make_async_copy(dst_hbm.at[pl.ds(row0, CH_CS + GRP)], didx,
                              ssem0).wait()

        off = cid * N_PAD

        def shift_body(j, carry):
            for c in range(CHUNK // 16):
                sl = pl.ds(c * 16, 16)
                sidx[j, sl] = sidx[j, sl] + off
            return carry

        lax.fori_loop(0, CH_CS + GRP, shift_body, 0)
        plsc.subcore_barrier()

        def gather(g, pset, sem):
            for b in range(GRP):
                pltpu.async_copy(u_hbm.at[sidx.at[g * GRP + b]],
                                 gbuf.at[pset * GRP + b], sem)

        def gather_wait(g, pset, sem):
            for b in range(GRP):
                pltpu.make_async_copy(u_hbm.at[sidx.at[g * GRP + b]],
                                      gbuf.at[pset * GRP + b], sem).wait()

        def scatter(g, pset, sem):
            for b in range(GRP):
                pltpu.async_copy(gbuf.at[pset * GRP + b],
                                 acc.at[didx.at[g * GRP + b]], sem, add=True)

        def scatter_wait(g, pset, sem):
            for b in range(GRP):
                pltpu.make_async_copy(gbuf.at[pset * GRP + b],
                                      acc.at[didx.at[g * GRP + b]],
                                      sem).wait()

        gather(0, 0, gsems[0])
        gather_wait(0, 0, gsems[0])
        scatter(0, 0, ssems[0])
        gather(1, 1, gsems[1])
        gather_wait(1, 1, gsems[1])
        scatter(1, 1, ssems[1])
        scatter_wait(0, 0, ssems[0])
        gather(2, 0, gsems[0])

        def pipe_body(t, carry):
            g0 = 2 * t
            gather_wait(g0, 0, gsems[0])
            scatter(g0, 0, ssems[0])
            scatter_wait(g0 - 1, 1, ssems[1])
            gather(g0 + 1, 1, gsems[1])
            g1 = g0 + 1
            gather_wait(g1, 1, gsems[1])
            scatter(g1, 1, ssems[1])
            scatter_wait(g1 - 1, 0, ssems[0])
            gather(g1 + 1, 0, gsems[0])
            return carry

        lax.fori_loop(1, NGRP // 2, pipe_body, 0)
        gather_wait(NGRP, 0, gsems[0])
        scatter_wait(NGRP - 1, 1, ssems[1])
        plsc.subcore_barrier()
        pltpu.sync_copy(acc.at[pl.ds(base, ROWS_PER_TILE)],
                        out_hbm.at[cid, pl.ds(base, ROWS_PER_TILE)])

    return k


_prop32 = _make_prop(32)
_prop16 = _make_prop(16)


@functools.partial(
    pl.kernel,
    mesh=_mesh,
    compiler_params=_SC_PARAMS,
    out_type=jax.ShapeDtypeStruct((NC, N_PAD), jnp.float32),
    scratch_types=[
        pltpu.VMEM((CH_DEG, CHUNK), jnp.int32),
        pltpu.VMEM((CHUNK,), jnp.float32),
        pltpu.VMEM((ROWS_PER_TILE,), jnp.float32),
        pltpu.VMEM_SHARED((N_PAD,), jnp.float32),
        pltpu.SemaphoreType.DMA,
    ],
)
def _deg_kernel(src_hbm, out_hbm, sidx, ones, zbuf, acc, ssem):
    cid = lax.axis_index("c")
    sid = lax.axis_index("s")
    w = cid * NS + sid
    base = sid * ROWS_PER_TILE

    for i in range(CHUNK // 16):
        ones[pl.ds(i * 16, 16)] = jnp.ones((16,), jnp.float32)
    for i in range(ROWS_PER_TILE // 16):
        zbuf[pl.ds(i * 16, 16)] = jnp.zeros((16,), jnp.float32)
    pltpu.sync_copy(zbuf, acc.at[pl.ds(base, ROWS_PER_TILE)])
    pltpu.sync_copy(src_hbm.at[pl.ds(w * CH_DEG, CH_DEG)], sidx)
    plsc.subcore_barrier()

    def grp_body(g, carry):
        ss = []
        for b in range(GRP):
            j = g * GRP + b
            ss.append(pltpu.async_copy(ones, acc.at[sidx.at[j]], ssem,
                                       add=True))
        for s in ss:
            s.wait()
        return carry

    lax.fori_loop(0, CH_DEG // GRP, grp_body, 0)
    plsc.subcore_barrier()
    pltpu.sync_copy(acc.at[pl.ds(base, ROWS_PER_TILE)],
                    out_hbm.at[cid, pl.ds(base, ROWS_PER_TILE)])


_TC_PARAMS = pltpu.CompilerParams(vmem_limit_bytes=100 * 1024 * 1024)


def _tc0_body(x_ref, wc_ref, o_y):
    o_y[...] = jnp.dot(x_ref[...], wc_ref[...],
                       preferred_element_type=jnp.float32,
                       precision=lax.Precision.HIGHEST)


_tc0 = pl.pallas_call(
    _tc0_body,
    compiler_params=_TC_PARAMS,
    out_shape=jax.ShapeDtypeStruct((N, 3 * HID), jnp.float32),
)


def _tc1_body(degp_ref, y_ref, o_dinv, o_y0, o_u):
    deg = degp_ref[0, :N] + degp_ref[1, :N]
    dinv = jnp.where(deg > 0, 1.0 / jnp.sqrt(deg), 0.0)
    o_dinv[...] = dinv[:, None]
    y = y_ref[...]
    o_y0[...] = y[:, 0:HID]
    d = dinv[:, None]
    z2 = jnp.zeros((N, 2), jnp.float32)
    o_u[0] = jnp.concatenate([d * y[:, HID:2 * HID], z2], axis=1)
    o_u[1] = jnp.concatenate([d * y[:, 2 * HID:3 * HID], z2], axis=1)


_tc1 = pl.pallas_call(
    _tc1_body,
    compiler_params=_TC_PARAMS,
    out_shape=(
        jax.ShapeDtypeStruct((N, 1), jnp.float32),
        jax.ShapeDtypeStruct((N, HID), jnp.float32),
        jax.ShapeDtypeStruct((NC, N, 32), jnp.float32),
    ),
)


def _tc2_body(r_ref, dinv_ref, o_p1, o_v):
    d = dinv_ref[...]
    r1 = r_ref[0][:N, 0:HID]
    r2 = r_ref[1][:N, 0:HID]
    o_p1[...] = -d * r1
    v2 = -(d * d) * r2
    o_v[0] = v2[:, 0:16]
    o_v[1] = jnp.concatenate([v2[:, 16:HID], jnp.zeros((N, 2), jnp.float32)],
                             axis=1)


_tc2 = pl.pallas_call(
    _tc2_body,
    compiler_params=_TC_PARAMS,
    out_shape=(
        jax.ShapeDtypeStruct((N, HID), jnp.float32),
        jax.ShapeDtypeStruct((NC, N, 16), jnp.float32),
    ),
)


def _combine_bn(r3_ref, y0_ref, p1_ref, dinv_ref, b_ref, g_ref, be_ref):
    r3 = jnp.concatenate([r3_ref[0][:N, 0:16], r3_ref[1][:N, 0:HID - 16]],
                         axis=1)
    q2 = -dinv_ref[...] * r3
    pre = y0_ref[...] + p1_ref[...] + 2.0 * q2 + b_ref[...]
    h = jnp.maximum(pre, 0.0)
    mu = jnp.mean(h, axis=0, keepdims=True)
    var = jnp.mean((h - mu) ** 2, axis=0, keepdims=True)
    return g_ref[...] * (h - mu) / jnp.sqrt(var + EPS) + be_ref[...]


def _tc3_body(hb_ref, wc2_ref, dinv_ref, o_y0b, o_u2):
    z = jnp.dot(hb_ref[...], wc2_ref[...], preferred_element_type=jnp.float32,
                precision=lax.Precision.HIGHEST)
    o_y0b[...] = z[:, 0:HID]
    d = dinv_ref[...]
    z2 = jnp.zeros((N, 2), jnp.float32)
    o_u2[0] = jnp.concatenate([d * z[:, HID:2 * HID], z2], axis=1)
    o_u2[1] = jnp.concatenate([d * z[:, 2 * HID:3 * HID], z2], axis=1)


_tc3 = pl.pallas_call(
    _tc3_body,
    compiler_params=_TC_PARAMS,
    out_shape=(
        jax.ShapeDtypeStruct((N, HID), jnp.float32),
        jax.ShapeDtypeStruct((NC, N, 32), jnp.float32),
    ),
)


def _tc5_body(r3_ref, y0_ref, p1_ref, dinv_ref, b_ref, g_ref, be_ref, o_out):
    o_out[...] = _combine_bn(r3_ref, y0_ref, p1_ref, dinv_ref, b_ref, g_ref,
                             be_ref)


_tc5 = pl.pallas_call(
    _tc5_body,
    compiler_params=_TC_PARAMS,
    out_shape=jax.ShapeDtypeStruct((N, HID), jnp.float32),
)


def _stack_pad(u):
    return jnp.pad(u, ((0, 0), (0, N_PAD - N), (0, 0))).reshape(
        2 * N_PAD, u.shape[2])


def kernel(x, edge_index, batch, W1, b1, W2, b2, gamma1, beta1, gamma2,
           beta2):
    src = edge_index[0]
    dst = edge_index[1]
    n_rows = E_PAD // CHUNK + GRP
    fill = jnp.full((n_rows * CHUNK - E,), N, jnp.int32)
    srcp = jnp.concatenate([src, fill]).reshape(n_rows, CHUNK)
    dstp = jnp.concatenate([dst, fill]).reshape(n_rows, CHUNK)

    degp = _deg_kernel(srcp)

    wc1 = jnp.concatenate([W1[0] - W1[2], W1[1], W1[2]], axis=1)
    wc2 = jnp.concatenate([W2[0] - W2[2], W2[1], W2[2]], axis=1)
    b1r = b1[None, :]
    b2r = b2[None, :]
    g1r = gamma1[None, :]
    g2r = gamma2[None, :]
    be1r = beta1[None, :]
    be2r = beta2[None, :]

    y_proj = _tc0(x, wc1)
    dinv, y0, u = _tc1(degp, y_proj)
    r = _prop32(_stack_pad(u), srcp, dstp)
    p1, v = _tc2(r, dinv)
    r3 = _prop16(_stack_pad(v), srcp, dstp)
    hb = _tc5(r3, y0, p1, dinv, b1r, g1r, be1r)
    y0b, u2 = _tc3(hb, wc2, dinv)
    r_2 = _prop32(_stack_pad(u2), srcp, dstp)
    p1b, v_2 = _tc2(r_2, dinv)
    r3b = _prop16(_stack_pad(v_2), srcp, dstp)
    return _tc5(r3b, y0b, p1b, dinv, b2r, g2r, be2r)
